# Initial kernel scaffold; baseline (speedup 1.0000x reference)
import jax
import jax.numpy as jnp
from jax.experimental import pallas as pl


def kernel(x, Wq, bq, Wk, bk, Wv, bv, Wc, bc, W1, b1, ln_g, ln_b, skip_scale, edge_index):
    raise NotImplementedError("write your pallas kernel here")


# speedup vs baseline: 3.6426x; 3.6426x over previous
import functools
import math

import jax
import jax.numpy as jnp
from jax import lax
from jax.experimental import pallas as pl
from jax.experimental.pallas import tpu as pltpu
from jax.experimental.pallas import tpu_sc as plsc

N = 10000
E = 160000
D = 256
HEADS = 4
HD = 64
HALF = 128
EPS = 1e-5

NC = 2
NS = 16

EPT = 10240
E_PAD = EPT * NS
BLK = 128
NBLK = EPT // BLK
NPAIR = NBLK // 2

_mesh = plsc.VectorSubcoreMesh(
    core_axis_name="c", subcore_axis_name="s", num_cores=NC, num_subcores=NS)
_params = pltpu.CompilerParams(needs_layout_passes=False)


def _k1_body(x_ref, wq_ref, bq_ref, wk_ref, bk_ref, wv_ref, bv_ref,
             xt_ref, q_ref, k_ref, v_ref):
  x = x_ref[...]
  nrm = jnp.sqrt(jnp.sum(x * x, axis=1, keepdims=True))
  nrm = jnp.maximum(nrm, 1e-10)
  z = jnp.minimum(nrm, 1.0 - EPS)
  at = 0.5 * jnp.log((1.0 + z) / (1.0 - z))
  xt = at * x / nrm
  xt_ref[...] = xt
  dn = (((1,), (1,)), ((), ()))
  q_ref[...] = lax.dot_general(xt, wq_ref[...], dn,
                               preferred_element_type=jnp.float32) + bq_ref[...]
  k_ref[...] = lax.dot_general(xt, wk_ref[...], dn,
                               preferred_element_type=jnp.float32) + bk_ref[...]
  v_ref[...] = lax.dot_general(xt, wv_ref[...], dn,
                               preferred_element_type=jnp.float32) + bv_ref[...]


def _k1(x, Wq, bq, Wk, bk, Wv, bv):
  nb = 10
  rb = N // nb
  full = pl.BlockSpec((D, D), lambda i: (0, 0))
  vecb = pl.BlockSpec((1, D), lambda i: (0, 0))
  rowb = pl.BlockSpec((rb, D), lambda i: (i, 0))
  return pl.pallas_call(
      _k1_body,
      grid=(nb,),
      in_specs=[rowb, full, vecb, full, vecb, full, vecb],
      out_specs=[rowb, rowb, rowb, rowb],
      out_shape=[jax.ShapeDtypeStruct((N, D), jnp.float32)] * 4,
  )(x, Wq, bq.reshape(1, D), Wk, bk.reshape(1, D), Wv, bv.reshape(1, D))


def _k2_body(q2, k2, src_hbm, dst_hbm, scores_hbm,
             src_all, dst_all, idxq0, idxk0, idxq1, idxk1,
             qb0, kb0, qb1, kb1, sbuf,
             semq0, semk0, semq1, semk1):
  c = lax.axis_index("c")
  s = lax.axis_index("s")
  ebase = s * EPT
  ii = lax.iota(jnp.int32, 16)
  pltpu.sync_copy(src_hbm.at[pl.ds(ebase, EPT)], src_all)
  pltpu.sync_copy(dst_hbm.at[pl.ds(ebase, EPT)], dst_all)

  def issue(j, idxq, idxk, qb, kb, semq, semk):
    for t in range(BLK // 16):
      sl = pl.ds(t * 16, 16)
      esl = pl.ds(j * BLK + t * 16, 16)
      idxq[sl] = dst_all[esl] * 2 + c
      idxk[sl] = src_all[esl] * 2 + c
    cq = pltpu.async_copy(q2.at[idxq], qb, semq)
    ck = pltpu.async_copy(k2.at[idxk], kb, semk)
    return cq, ck

  def compute(j, qb, kb):
    @pl.loop(0, BLK // 16)
    def _g(g):
      rows = ii + g * 16
      sA = jnp.zeros((16,), jnp.float32)
      sB = jnp.zeros((16,), jnp.float32)
      for d in range(HALF):
        cd = jnp.full((16,), d, jnp.int32)
        p = (plsc.load_gather(qb, [rows, cd])
             * plsc.load_gather(kb, [rows, cd]))
        if d < HD:
          sA = sA + p
        else:
          sB = sB + p
      sbuf[0, pl.ds(j * BLK + g * 16, 16)] = sA
      sbuf[1, pl.ds(j * BLK + g * 16, 16)] = sB

  @pl.loop(0, NPAIR)
  def _pair(jj):
    j0 = jj * 2
    j1 = jj * 2 + 1
    c0 = issue(j0, idxq0, idxk0, qb0, kb0, semq0, semk0)
    c1 = issue(j1, idxq1, idxk1, qb1, kb1, semq1, semk1)
    c0[0].wait()
    c0[1].wait()
    compute(j0, qb0, kb0)
    c1[0].wait()
    c1[1].wait()
    compute(j1, qb1, kb1)

  pltpu.sync_copy(sbuf.at[0], scores_hbm.at[pl.ds(2 * c * E_PAD + ebase, EPT)])
  pltpu.sync_copy(sbuf.at[1],
                  scores_hbm.at[pl.ds((2 * c + 1) * E_PAD + ebase, EPT)])


def _k2(q2, k2, src_pad, dst_pad):
  return pl.kernel(
      _k2_body,
      out_type=jax.ShapeDtypeStruct((HEADS * E_PAD,), jnp.float32),
      mesh=_mesh,
      compiler_params=_params,
      scratch_types=[
          pltpu.VMEM((EPT,), jnp.int32),
          pltpu.VMEM((EPT,), jnp.int32),
          pltpu.VMEM((BLK,), jnp.int32),
          pltpu.VMEM((BLK,), jnp.int32),
          pltpu.VMEM((BLK,), jnp.int32),
          pltpu.VMEM((BLK,), jnp.int32),
          pltpu.VMEM((BLK, HALF), jnp.float32),
          pltpu.VMEM((BLK, HALF), jnp.float32),
          pltpu.VMEM((BLK, HALF), jnp.float32),
          pltpu.VMEM((BLK, HALF), jnp.float32),
          pltpu.VMEM((2, EPT), jnp.float32),
          pltpu.SemaphoreType.DMA,
          pltpu.SemaphoreType.DMA,
          pltpu.SemaphoreType.DMA,
          pltpu.SemaphoreType.DMA,
      ],
  )(q2, k2, src_pad, dst_pad)


def _k3_body(s_ref, w_ref):
  srs = s_ref[...] * (1.0 / math.sqrt(HD))
  col = lax.broadcasted_iota(jnp.int32, (HEADS, E_PAD), 1)
  valid = col < E
  m = jnp.max(jnp.where(valid, srs, -1e30), axis=1, keepdims=True)
  e = jnp.where(valid, jnp.exp(srs - m), 0.0)
  zsum = jnp.sum(e, axis=1, keepdims=True)
  w_ref[...] = e / zsum


def _k3(scores):
  return pl.pallas_call(
      _k3_body,
      out_shape=jax.ShapeDtypeStruct((HEADS, E_PAD), jnp.float32),
  )(scores)


def _k4_body(v2, src_hbm, dst_hbm, w_hbm, msg_hbm,
             srcp, dstp, wp, idxv0, dstb0, idxv1, dstb1,
             vb0, vb1, zbuf, acc, semm, semv0, semv1, sems0, sems1):
  c = lax.axis_index("c")
  s = lax.axis_index("s")
  ebase = s * EPT
  ii = lax.iota(jnp.int32, 16)
  z16 = jnp.zeros((16,), jnp.float32)

  @pl.loop(0, 40)
  def _zr(r):
    for t in range(HALF // 16):
      zbuf[r, pl.ds(t * 16, 16)] = z16

  @pl.when(s < 10)
  def _zero():
    @pl.loop(0, 25)
    def _zc(kk):
      pltpu.sync_copy(zbuf, acc.at[pl.ds(s * 1000 + kk * 40, 40)])
  plsc.subcore_barrier()

  PAIR = 2 * BLK

  def scale(vb, woff):
    @pl.loop(0, BLK // 16)
    def _g(g):
      rows = ii + g * 16
      wA = wp[0, pl.ds(woff + g * 16, 16)]
      wB = wp[1, pl.ds(woff + g * 16, 16)]
      for d in range(HALF):
        cd = jnp.full((16,), d, jnp.int32)
        col = plsc.load_gather(vb, [rows, cd])
        col = col * (wA if d < HD else wB)
        plsc.store_scatter(vb, [rows, cd], col)

  @pl.loop(0, NPAIR)
  def _pair(jj):
    poff = ebase + jj * PAIR
    m0 = pltpu.async_copy(src_hbm.at[pl.ds(poff, PAIR)], srcp, semm)
    m1 = pltpu.async_copy(dst_hbm.at[pl.ds(poff, PAIR)], dstp, semm)
    m2 = pltpu.async_copy(
        w_hbm.at[pl.ds(2 * c * E_PAD + poff, PAIR)], wp.at[0], semm)
    m3 = pltpu.async_copy(
        w_hbm.at[pl.ds((2 * c + 1) * E_PAD + poff, PAIR)], wp.at[1], semm)
    m0.wait()
    m1.wait()
    m2.wait()
    m3.wait()
    for t in range(BLK // 16):
      sl = pl.ds(t * 16, 16)
      idxv0[sl] = srcp[sl] * 2 + c
      dstb0[sl] = dstp[sl]
      sl2 = pl.ds(BLK + t * 16, 16)
      idxv1[sl] = srcp[sl2] * 2 + c
      dstb1[sl] = dstp[sl2]
    g0 = pltpu.async_copy(v2.at[idxv0], vb0, semv0)
    g1 = pltpu.async_copy(v2.at[idxv1], vb1, semv1)
    g0.wait()
    scale(vb0, 0)
    s0 = pltpu.async_copy(vb0, acc.at[dstb0], sems0, add=True)
    g1.wait()
    scale(vb1, BLK)
    s1 = pltpu.async_copy(vb1, acc.at[dstb1], sems1, add=True)
    s0.wait()
    s1.wait()

  plsc.subcore_barrier()

  @pl.when(s < 10)
  def _out():
    pltpu.sync_copy(acc.at[pl.ds(s * 1000, 1000)],
                    msg_hbm.at[c, pl.ds(s * 1000, 1000)])


def _k4(v2, src_pad, dst_pad, w):
  return pl.kernel(
      _k4_body,
      out_type=jax.ShapeDtypeStruct((NC, N, HALF), jnp.float32),
      mesh=_mesh,
      compiler_params=_params,
      scratch_types=[
          pltpu.VMEM((2 * BLK,), jnp.int32),
          pltpu.VMEM((2 * BLK,), jnp.int32),
          pltpu.VMEM((2, 2 * BLK), jnp.float32),
          pltpu.VMEM((BLK,), jnp.int32),
          pltpu.VMEM((BLK,), jnp.int32),
          pltpu.VMEM((BLK,), jnp.int32),
          pltpu.VMEM((BLK,), jnp.int32),
          pltpu.VMEM((BLK, HALF), jnp.float32),
          pltpu.VMEM((BLK, HALF), jnp.float32),
          pltpu.VMEM((40, HALF), jnp.float32),
          pltpu.VMEM_SHARED((N, HALF), jnp.float32),
          pltpu.SemaphoreType.DMA,
          pltpu.SemaphoreType.DMA,
          pltpu.SemaphoreType.DMA,
          pltpu.SemaphoreType.DMA,
          pltpu.SemaphoreType.DMA,
      ],
  )(v2, src_pad, dst_pad, w)


def _k5_body(xt_ref, msg_ref, wc_ref, bc_ref, w1a_ref, w1b_ref, b1_ref,
             g_ref, b_ref, skip_ref, out_ref):
  xt = xt_ref[...]
  dn = (((1,), (1,)), ((), ()))
  msgs = lax.dot_general(msg_ref[...], wc_ref[...], dn,
                         preferred_element_type=jnp.float32) + bc_ref[...]
  h = (lax.dot_general(xt, w1a_ref[...], dn,
                       preferred_element_type=jnp.float32)
       + lax.dot_general(msgs, w1b_ref[...], dn,
                         preferred_element_type=jnp.float32)
       + b1_ref[...])
  mu = jnp.mean(h, axis=1, keepdims=True)
  var = jnp.mean(jnp.square(h - mu), axis=1, keepdims=True)
  h = (h - mu) / jnp.sqrt(var + 1e-5) * g_ref[...] + b_ref[...]
  h = h * jax.nn.sigmoid(h)
  upd = h + skip_ref[0, 0] * xt
  un = jnp.sqrt(jnp.sum(upd * upd, axis=1, keepdims=True))
  un = jnp.maximum(un, 1e-10)
  ex = jnp.tanh(un) * upd / un
  exn = jnp.sqrt(jnp.sum(ex * ex, axis=1, keepdims=True))
  exn = jnp.maximum(exn, 1e-10)
  maxn = 1.0 - EPS
  out_ref[...] = jnp.where(exn > maxn, ex / exn * maxn, ex)


def _k5(xt, msg, Wc, bc, W1, b1, ln_g, ln_b, skip_scale):
  nb = 10
  rb = N // nb
  rowb = pl.BlockSpec((rb, D), lambda i: (i, 0))
  full = pl.BlockSpec((D, D), lambda i: (0, 0))
  vecb = pl.BlockSpec((1, D), lambda i: (0, 0))
  oneb = pl.BlockSpec((1, 1), lambda i: (0, 0))
  return pl.pallas_call(
      _k5_body,
      grid=(nb,),
      in_specs=[rowb, rowb, full, vecb, full, full, vecb, vecb, vecb, oneb],
      out_specs=rowb,
      out_shape=jax.ShapeDtypeStruct((N, D), jnp.float32),
  )(xt, msg, Wc, bc.reshape(1, D), W1[:, :D], W1[:, D:], b1.reshape(1, D),
    ln_g.reshape(1, D), ln_b.reshape(1, D), skip_scale.reshape(1, 1))


@jax.jit
def kernel(x, Wq, bq, Wk, bk, Wv, bv, Wc, bc, W1, b1, ln_g, ln_b,
           skip_scale, edge_index):
  xt, q, k, v = _k1(x, Wq, bq, Wk, bk, Wv, bv)
  q2 = q.reshape(2 * N, HALF)
  k2 = k.reshape(2 * N, HALF)
  v2 = v.reshape(2 * N, HALF)

  src = edge_index[0].astype(jnp.int32)
  dst = edge_index[1].astype(jnp.int32)
  pad = E_PAD - E
  src_pad = jnp.pad(src, (0, pad))
  dst_pad = jnp.pad(dst, (0, pad))

  scores = _k2(q2, k2, src_pad, dst_pad).reshape(HEADS, E_PAD)
  w = _k3(scores).reshape(HEADS * E_PAD)
  msg2 = _k4(v2, src_pad, dst_pad, w)
  msg = jnp.concatenate([msg2[0], msg2[1]], axis=1)
  return _k5(xt, msg, Wc, bc, W1, b1, ln_g, ln_b, skip_scale)
